# baseline (speedup 1.0000x reference)
import functools

import jax
import jax.numpy as jnp
from jax import lax
from jax.experimental import pallas as pl
from jax.experimental.pallas import tpu as pltpu
from jax.experimental.pallas import tpu_sc as plsc

_NC = 2
_NS = 16
_L = 16
_NW = _NC * _NS


def kernel(output_idx, output, history, assist_rate, assist_weight):
    B, NO = output.shape
    bpw = B // _NW
    groups = bpw // _L
    mesh = plsc.VectorSubcoreMesh(core_axis_name="c", subcore_axis_name="s")

    @functools.partial(
        pl.kernel,
        out_type=jax.ShapeDtypeStruct((B,), jnp.float32),
        mesh=mesh,
        scratch_types=[
            pltpu.VMEM((bpw * NO,), jnp.float32),
            pltpu.VMEM((bpw,), jnp.int32),
            pltpu.VMEM((bpw,), jnp.float32),
            pltpu.VMEM((bpw,), jnp.float32),
            pltpu.VMEM((bpw,), jnp.float32),
            pltpu.VMEM((2 * _L,), jnp.float32),
            pltpu.SemaphoreType.DMA,
            pltpu.SemaphoreType.DMA,
        ],
        compiler_params=pltpu.CompilerParams(needs_layout_passes=False),
    )
    def _assist_sc(idx_hbm, mat_hbm, hist_hbm, rate_hbm, w_hbm, tgt_hbm,
                   mat_v, idx_v, ar_v, hist_v, res_v, w_v, sem_mat, sem_ar):
        wid = lax.axis_index("s") * _NC + lax.axis_index("c")
        base = wid * bpw

        cp_mat = pltpu.async_copy(mat_hbm.at[pl.ds(base * NO, bpw * NO)], mat_v, sem_mat)
        pltpu.sync_copy(idx_hbm.at[pl.ds(base, bpw)], idx_v)
        cp_ar = pltpu.async_copy(rate_hbm.at[idx_v], ar_v, sem_ar)
        pltpu.sync_copy(hist_hbm.at[pl.ds(base, bpw)], hist_v)

        w_v[pl.ds(_L, _L)] = jnp.full((_L,), -jnp.inf, jnp.float32)
        pltpu.sync_copy(w_hbm, w_v.at[pl.ds(0, NO)])
        w0 = w_v[pl.ds(0, _L)]
        w1 = w_v[pl.ds(_L, _L)]
        wraw = [w0[o] if o < _L else w1[o - _L] for o in range(NO)]
        m = functools.reduce(jnp.maximum, wraw)
        e0 = jnp.exp(w0 - m)
        e1 = jnp.exp(w1 - m)
        es = [e0[o] if o < _L else e1[o - _L] for o in range(NO)]
        den = lax.broadcast(functools.reduce(jnp.add, es), (_L,))
        wn0 = e0 / den
        wn1 = e1 / den
        ws = [wn0[o] if o < _L else wn1[o - _L] for o in range(NO)]

        cp_mat.wait()
        cp_ar.wait()

        lanes_no = lax.iota(jnp.int32, _L) * NO

        def body(g, carry):
            off = g * _L
            flat0 = lanes_no + off * NO
            acc = jnp.zeros((_L,), jnp.float32)
            for o in range(NO):
                acc = acc + plsc.load_gather(mat_v, [flat0 + o]) * ws[o]
            res_v[pl.ds(off, _L)] = (hist_v[pl.ds(off, _L)]
                                     + ar_v[pl.ds(off, _L)] * acc)
            return carry

        lax.fori_loop(0, groups, body, 0)
        pltpu.sync_copy(res_v, tgt_hbm.at[pl.ds(base, bpw)])

    return _assist_sc(output_idx.astype(jnp.int32), output.reshape(-1),
                      history, assist_rate, assist_weight)
